# Initial kernel scaffold; baseline (speedup 1.0000x reference)
import jax
import jax.numpy as jnp
from jax.experimental import pallas as pl


def kernel(input_tensor, token_type_ids, token_type_table, full_position_embeddings, ln_gamma, ln_beta):
    raise NotImplementedError("write your pallas kernel here")


# speedup vs baseline: 2.6702x; 2.6702x over previous
import functools

import jax
import jax.numpy as jnp
from jax import lax
from jax.experimental import pallas as pl
from jax.experimental.pallas import tpu as pltpu

B, S, H = 64, 512, 768
TT_VOCAB = 16
LN_EPS = 1e-05
BB = 4


def _fused_body(x_ref, ids_ref, tt_ref, pos_ref, g_ref, b_ref, o_ref):
    tt_tab = tt_ref[...]
    pos = pos_ref[...]
    gamma = g_ref[...]
    beta = b_ref[...]
    for bb in range(BB):
        ids = ids_ref[bb]
        iota = lax.broadcasted_iota(jnp.int32, (S, TT_VOCAB), 1)
        onehot = (ids == iota).astype(jnp.float32)
        tt = jnp.dot(onehot, tt_tab, preferred_element_type=jnp.float32)
        y = x_ref[bb] + tt + pos
        mean = jnp.mean(y, axis=-1, keepdims=True)
        c = y - mean
        var = jnp.mean(c * c, axis=-1, keepdims=True)
        o_ref[bb] = c * lax.rsqrt(var + LN_EPS) * gamma + beta


@jax.jit
def _fused(input_tensor, ids3d, token_type_table, pos, gamma2d, beta2d):
    grid = (B // BB,)
    return pl.pallas_call(
        _fused_body,
        grid=grid,
        in_specs=[
            pl.BlockSpec((BB, S, H), lambda i: (i, 0, 0)),
            pl.BlockSpec((BB, S, 1), lambda i: (i, 0, 0)),
            pl.BlockSpec((TT_VOCAB, H), lambda i: (0, 0)),
            pl.BlockSpec((S, H), lambda i: (0, 0)),
            pl.BlockSpec((1, H), lambda i: (0, 0)),
            pl.BlockSpec((1, H), lambda i: (0, 0)),
        ],
        out_specs=pl.BlockSpec((BB, S, H), lambda i: (i, 0, 0)),
        out_shape=jax.ShapeDtypeStruct((B, S, H), jnp.float32),
        compiler_params=pltpu.CompilerParams(
            dimension_semantics=("arbitrary",),
        ),
    )(input_tensor, ids3d, token_type_table, pos, gamma2d, beta2d)


def kernel(input_tensor, token_type_ids, token_type_table, full_position_embeddings, ln_gamma, ln_beta):
    ids3d = token_type_ids.reshape(B, S, 1)
    pos = full_position_embeddings[:S]
    return _fused(
        input_tensor,
        ids3d,
        token_type_table,
        pos,
        ln_gamma.reshape(1, H),
        ln_beta.reshape(1, H),
    )
